# baseline (speedup 1.0000x reference)
import functools

import jax
import jax.numpy as jnp
from jax import lax
from jax.experimental import pallas as pl
from jax.experimental.pallas import tpu as pltpu
from jax.experimental.pallas import tpu_sc as plsc

_B = 16
_S = 2048
_D = 256
_TOTAL = 16384
_HALF = _S // 2
_C = 128
_NCHUNK = _HALF // _C
_CD = _C * _D
_L = 16


def _body(flat, cu, out, cu_v, zbuf, sem_z, sem_d):
    cid = lax.axis_index("c")
    sid = lax.axis_index("s")
    wid = cid * 16 + sid
    b = wid // 2
    h = wid % 2

    pltpu.sync_copy(cu, cu_v)
    start = cu_v[pl.ds(b, 16)][0]
    ln = cu_v[pl.ds(b + 1, 16)][0] - start
    t = jnp.clip(ln - h * _HALF, 0, _HALF)
    base = b * _S + h * _HALF
    floor_t = (t // _C) * _C
    nz = (_HALF - floor_t) // _C
    nd = (t + _C - 1) // _C

    def _zero(i, _):
        zbuf[pl.ds(i * _L, _L)] = jnp.zeros((_L,), jnp.float32)
        return 0
    lax.fori_loop(0, _CD // _L, _zero, 0)

    for j in range(_NCHUNK):
        @pl.when(j < nz)
        def _():
            row = base + floor_t + j * _C
            pltpu.async_copy(zbuf, out.at[pl.ds(row * _D, _CD)], sem_z)
    for j in range(_NCHUNK):
        @pl.when(j < nz)
        def _():
            row = base + floor_t + j * _C
            pltpu.make_async_copy(zbuf, out.at[pl.ds(row * _D, _CD)], sem_z).wait()

    for i in range(_NCHUNK):
        @pl.when(i < nd)
        def _():
            off = jnp.minimum(i * _C, t - _C)
            src = (start + h * _HALF + off) * _D
            dst = (base + off) * _D
            pltpu.async_copy(flat.at[pl.ds(src, _CD)], out.at[pl.ds(dst, _CD)], sem_d)
    for i in range(_NCHUNK):
        @pl.when(i < nd)
        def _():
            off = jnp.minimum(i * _C, t - _C)
            src = (start + h * _HALF + off) * _D
            dst = (base + off) * _D
            pltpu.make_async_copy(flat.at[pl.ds(src, _CD)], out.at[pl.ds(dst, _CD)], sem_d).wait()


def kernel(flat, cu_seqlens):
    cu = jnp.concatenate(
        [cu_seqlens.astype(jnp.int32), jnp.zeros((23,), jnp.int32)])
    mesh = plsc.VectorSubcoreMesh(core_axis_name="c", subcore_axis_name="s")
    run = pl.kernel(
        _body,
        out_type=jax.ShapeDtypeStruct((_B * _S * _D,), jnp.float32),
        mesh=mesh,
        scratch_types=[
            pltpu.VMEM((40,), jnp.int32),
            pltpu.VMEM((_CD,), jnp.float32),
            pltpu.SemaphoreType.DMA,
            pltpu.SemaphoreType.DMA,
        ],
    )
    out = run(flat.reshape(_TOTAL * _D), cu)
    return out.reshape(_B, _S, _D)


# speedup vs baseline: 6.5113x; 6.5113x over previous
import jax
import jax.numpy as jnp
from jax import lax
from jax.experimental import pallas as pl
from jax.experimental.pallas import tpu as pltpu
from jax.experimental.pallas import tpu_sc as plsc

_B = 16
_S = 2048
_D = 256
_TOTAL = 16384
_HALF = _S // 2
_C = 128
_NCHUNK = _HALF // _C
_CD = _C * _D
_L = 16


def _body(flat, cu, out, cu_v, zbuf, bufa, bufb,
          sem_c, sem_z, sem_ia, sem_ib, sem_oa, sem_ob):
    cid = lax.axis_index("c")
    sid = lax.axis_index("s")
    wid = cid * 16 + sid
    b = wid // 2
    h = wid % 2

    pltpu.async_copy(cu, cu_v, sem_c)

    def _zero(i, _):
        zbuf[pl.ds(i * _L, _L)] = jnp.zeros((_L,), jnp.float32)
        return 0
    lax.fori_loop(0, _CD // _L, _zero, 0, unroll=8)

    pltpu.make_async_copy(cu, cu_v, sem_c).wait()
    start = cu_v[pl.ds(b, 16)][0]
    ln = cu_v[pl.ds(b + 1, 16)][0] - start
    t = jnp.clip(ln - h * _HALF, 0, _HALF)
    base = b * _S + h * _HALF
    floor_t = (t // _C) * _C
    nz = (_HALF - floor_t) // _C
    nd = (t + _C - 1) // _C

    def _src(i):
        off = jnp.minimum(i * _C, t - _C)
        return (start + h * _HALF + off) * _D

    def _dst(i):
        off = jnp.minimum(i * _C, t - _C)
        return (base + off) * _D

    bufs = (bufa, bufb)
    isems = (sem_ia, sem_ib)
    osems = (sem_oa, sem_ob)

    for j in range(_NCHUNK):
        @pl.when(j < nz)
        def _():
            row = base + floor_t + j * _C
            pltpu.async_copy(zbuf, out.at[pl.ds(row * _D, _CD)], sem_z)

    @pl.when(0 < nd)
    def _():
        pltpu.async_copy(flat.at[pl.ds(_src(0), _CD)], bufa, sem_ia)

    for j in range(_NCHUNK):
        @pl.when(j < nz)
        def _():
            pltpu.make_async_copy(zbuf, out.at[pl.ds(base * _D, _CD)], sem_z).wait()

    for i in range(_NCHUNK):
        @pl.when(i < nd)
        def _():
            pltpu.make_async_copy(
                flat.at[pl.ds(_src(i), _CD)], bufs[i % 2], isems[i % 2]).wait()
            pltpu.async_copy(bufs[i % 2], out.at[pl.ds(_dst(i), _CD)], osems[i % 2])
        @pl.when(i + 1 < nd)
        def _():
            if i >= 1:
                pltpu.make_async_copy(
                    bufs[(i - 1) % 2], out.at[pl.ds(base * _D, _CD)],
                    osems[(i - 1) % 2]).wait()
            pltpu.async_copy(
                flat.at[pl.ds(_src(i + 1), _CD)], bufs[(i + 1) % 2],
                isems[(i + 1) % 2])

    @pl.when(nd >= 2)
    def _():
        pltpu.make_async_copy(bufa, out.at[pl.ds(base * _D, _CD)], sem_oa).wait()
        pltpu.make_async_copy(bufb, out.at[pl.ds(base * _D, _CD)], sem_ob).wait()
    @pl.when(nd == 1)
    def _():
        pltpu.make_async_copy(bufa, out.at[pl.ds(base * _D, _CD)], sem_oa).wait()


def kernel(flat, cu_seqlens):
    cu = jnp.concatenate(
        [cu_seqlens.astype(jnp.int32), jnp.zeros((23,), jnp.int32)])
    mesh = plsc.VectorSubcoreMesh(core_axis_name="c", subcore_axis_name="s")
    run = pl.kernel(
        _body,
        out_type=jax.ShapeDtypeStruct((_B * _S * _D,), jnp.float32),
        mesh=mesh,
        scratch_types=[
            pltpu.VMEM((40,), jnp.int32),
            pltpu.VMEM((_CD,), jnp.float32),
            pltpu.VMEM((_CD,), jnp.float32),
            pltpu.VMEM((_CD,), jnp.float32),
            pltpu.SemaphoreType.DMA,
            pltpu.SemaphoreType.DMA,
            pltpu.SemaphoreType.DMA,
            pltpu.SemaphoreType.DMA,
            pltpu.SemaphoreType.DMA,
            pltpu.SemaphoreType.DMA,
        ],
    )
    out = run(flat.reshape(_TOTAL * _D), cu)
    return out.reshape(_B, _S, _D)
